# baseline (speedup 1.0000x reference)
import dataclasses
import functools

import jax
import jax.numpy as jnp
from jax import lax
from jax.experimental import pallas as pl
from jax.experimental.pallas import tpu as pltpu
from jax.experimental.pallas import tpu_sc as plsc

N_NODES = 10000
N_EDGES = 320000
N_HID = 128
N_OUT = 64
N_LAYERS = 2
N_GRAPHS = 64
ATOM_FEATS = 9
ATOM_VOCAB = 119

NC, NS = 2, 16
NW = NC * NS
NP = 10240
NPW = NP // NW
NPA = 10112
RPT = NPA // NS
K = 64
CPW = 160
EP = NW * CPW * K
NG = EP // K
CPW0 = CPW
CPW1 = 2 * CPW - CPW0
NODE_CHUNK = 64
ENC_CHUNKS = NPW // NODE_CHUNK


def _mesh():
    return plsc.VectorSubcoreMesh(core_axis_name="c", subcore_axis_name="s")


def _sc_params(**kw):
    cp = pltpu.CompilerParams()
    if "needs_layout_passes" in pltpu.CompilerParams.__dataclass_fields__:
        cp = dataclasses.replace(cp, needs_layout_passes=False)
    for k, v in kw.items():
        if k in pltpu.CompilerParams.__dataclass_fields__:
            cp = dataclasses.replace(cp, **{k: v})
    return cp


def _encoder(flat_emb, attr_c):
    @functools.partial(
        pl.kernel,
        mesh=_mesh(),
        out_type=jax.ShapeDtypeStruct((NP, N_HID), jnp.float32),
        scratch_types=[
            pltpu.VMEM((ATOM_FEATS, NODE_CHUNK), jnp.int32),
            pltpu.VMEM((ATOM_FEATS, NODE_CHUNK, N_HID), jnp.float32),
            pltpu.VMEM((NODE_CHUNK, N_HID), jnp.float32),
            pltpu.SemaphoreType.DMA,
        ],
        compiler_params=_sc_params(),
    )
    def enc(emb_hbm, attr_hbm, out_hbm, ibuf, rbuf, acc, sem):
        cid = lax.axis_index("c")
        sid = lax.axis_index("s")
        wid = sid * NC + cid
        gbase = wid * ENC_CHUNKS

        @pl.loop(0, ENC_CHUNKS)
        def _(c):
            pltpu.sync_copy(attr_hbm.at[gbase + c], ibuf)
            for f in range(1, ATOM_FEATS):
                for t in range(NODE_CHUNK // 16):
                    sl = pl.ds(t * 16, 16)
                    ibuf[f, sl] = ibuf[f, sl] + (ATOM_VOCAB * f)
            for f in range(ATOM_FEATS):
                pltpu.async_copy(emb_hbm.at[ibuf.at[f]], rbuf.at[f], sem)
            for f in range(ATOM_FEATS):
                pltpu.make_async_copy(emb_hbm.at[ibuf.at[f]], rbuf.at[f],
                                      sem).wait()

            @pl.loop(0, NODE_CHUNK)
            def _(r):
                for j in range(N_HID // 16):
                    sl = pl.ds(j * 16, 16)
                    s = rbuf[0, r, sl]
                    for f in range(1, ATOM_FEATS):
                        s = s + rbuf[f, r, sl]
                    acc[r, sl] = s

            pltpu.sync_copy(
                acc, out_hbm.at[pl.ds((gbase + c) * NODE_CHUNK, NODE_CHUNK)])

    return enc(flat_emb, attr_c)


def _spmm(nrep_bf, edata):
    @functools.partial(
        pl.kernel,
        mesh=_mesh(),
        out_type=(
            jax.ShapeDtypeStruct((NC, NP, N_HID), jnp.float32),
            jax.ShapeDtypeStruct((NC, NP), jnp.float32),
        ),
        scratch_types=[
            pltpu.VMEM((3, K), jnp.int32),
            pltpu.VMEM((3, K), jnp.int32),
            pltpu.VMEM((K,), jnp.float32),
            pltpu.VMEM((K,), jnp.float32),
            pltpu.VMEM((K, N_HID // 2), jnp.int32),
            pltpu.VMEM((K, N_HID // 2), jnp.int32),
            pltpu.VMEM((K, N_HID), jnp.float32),
            pltpu.VMEM((K, N_HID), jnp.float32),
            pltpu.VMEM_SHARED((NPA, N_HID), jnp.float32),
            pltpu.VMEM_SHARED((NPA,), jnp.float32),
            pltpu.SemaphoreType.DMA,
            pltpu.SemaphoreType.DMA,
            pltpu.SemaphoreType.DMA,
            pltpu.SemaphoreType.DMA,
        ],
        compiler_params=_sc_params(use_tc_tiling_on_sc=False),
    )
    def spmm(nrep_hbm, edata_hbm, wsum_hbm, den_hbm,
             ebuf0, ebuf1, av0, av1, rb0, rb1, rows0, rows1,
             wsum_sh, den_sh, semg0, semg1, sems0, sems1):
        cid = lax.axis_index("c")
        sid = lax.axis_index("s")
        is0 = cid == 0
        gbase = jnp.where(is0, sid * CPW0, NS * CPW0 + sid * CPW1)
        nch = jnp.where(is0, CPW0, CPW1)

        ebufs = (ebuf0, ebuf1)
        avs = (av0, av1)
        rbs = (rb0, rb1)
        rows = (rows0, rows1)
        semgs = (semg0, semg1)
        semss = (sems0, sems1)

        scope_zero = jax.named_scope("spmm_zero")
        scope_zero.__enter__()
        zero16 = jnp.zeros((16,), jnp.float32)

        @pl.loop(0, K)
        def _(r):
            for j in range(N_HID // 16):
                rows0[r, pl.ds(j * 16, 16)] = zero16

        for j in range(K // 16):
            av0[pl.ds(j * 16, 16)] = zero16

        stripe = sid * RPT

        for c in range(10):
            sz = K if c < 9 else RPT - 9 * K
            pltpu.sync_copy(rows0.at[pl.ds(0, sz)],
                            wsum_sh.at[pl.ds(stripe + c * K, sz)])
            pltpu.sync_copy(av0.at[pl.ds(0, sz)],
                            den_sh.at[pl.ds(stripe + c * K, sz)])

        @pl.when(sid == NS - 1)
        def _():
            for c in range((NP - NPA) // K):
                pltpu.sync_copy(rows0,
                                wsum_hbm.at[cid, pl.ds(NPA + c * K, K)])
                pltpu.sync_copy(av0, den_hbm.at[cid, pl.ds(NPA + c * K, K)])

        plsc.subcore_barrier()
        scope_zero.__exit__(None, None, None)

        def load_idx_and_a(b, g):
            pltpu.sync_copy(edata_hbm.at[g], ebufs[b])
            for j in range(K // 16):
                sl = pl.ds(j * 16, 16)
                avs[b][sl] = plsc.bitcast(ebufs[b][2, sl], jnp.float32)

        def start_gather(b):
            pltpu.async_copy(nrep_hbm.at[ebufs[b].at[0]], rbs[b], semgs[b])

        def wait_gather(b):
            pltpu.make_async_copy(nrep_hbm.at[ebufs[b].at[0]], rbs[b],
                                  semgs[b]).wait()

        def start_scatter(b):
            pltpu.async_copy(rows[b], wsum_sh.at[ebufs[b].at[1]], semss[b],
                             add=True)
            pltpu.async_copy(avs[b], den_sh.at[ebufs[b].at[1]], semss[b],
                             add=True)

        def wait_scatter(b):
            pltpu.make_async_copy(rows[b], wsum_sh.at[ebufs[b].at[1]],
                                  semss[b]).wait()
            pltpu.make_async_copy(avs[b], den_sh.at[ebufs[b].at[1]],
                                  semss[b]).wait()

        himask = jnp.full((16,), -65536, jnp.int32)

        def scale(b):
            @pl.loop(0, K)
            def _(k):
                vs = plsc.load_gather(avs[b], [jnp.full((16,), k, jnp.int32)])
                for g in range(N_HID // 32):
                    w = rbs[b][k, pl.ds(g * 16, 16)]
                    lo = plsc.bitcast(w << 16, jnp.float32) * vs
                    hi = plsc.bitcast(w & himask, jnp.float32) * vs
                    rows[b][k, pl.ds(g * 32, 16)] = lo
                    rows[b][k, pl.ds(g * 32 + 16, 16)] = hi

        scope_loop = jax.named_scope("spmm_loop")
        scope_loop.__enter__()
        load_idx_and_a(0, gbase)
        start_gather(0)

        def body(i, carry):
            c = i * 2
            wait_gather(0)

            @pl.when(c >= 2)
            def _():
                wait_scatter(1)

            load_idx_and_a(1, gbase + c + 1)
            start_gather(1)
            scale(0)
            start_scatter(0)

            wait_gather(1)
            wait_scatter(0)

            @pl.when(c + 2 < nch)
            def _():
                load_idx_and_a(0, gbase + c + 2)
                start_gather(0)

            scale(1)
            start_scatter(1)
            return carry

        lax.fori_loop(0, nch // 2, body, 0)

        wait_scatter(1)
        plsc.subcore_barrier()
        scope_loop.__exit__(None, None, None)

        scope_out = jax.named_scope("spmm_out")
        scope_out.__enter__()

        for c in range(10):
            sz = K if c < 9 else RPT - 9 * K
            off = stripe + c * K
            pltpu.sync_copy(wsum_sh.at[pl.ds(off, sz)],
                            wsum_hbm.at[cid, pl.ds(off, sz)])
            pltpu.sync_copy(den_sh.at[pl.ds(off, sz)],
                            den_hbm.at[cid, pl.ds(off, sz)])

        scope_out.__exit__(None, None, None)

    return spmm(nrep_bf, edata)


def _dense_body(w_ref, d_ref, x_ref, W_ref, b_ref, g_ref, bb_ref, o_ref):
    ws = w_ref[0] + w_ref[1]
    den = d_ref[0] + d_ref[1]
    aggr = ws * (1.0 / jnp.maximum(den, 1e-30))
    h = jnp.dot(jnp.maximum(aggr, 0.0), W_ref[...],
                preferred_element_type=jnp.float32) + b_ref[...]
    x = h + x_ref[...]
    mean = jnp.mean(x, axis=1, keepdims=True)
    xc = x - mean
    var = jnp.mean(xc * xc, axis=1, keepdims=True)
    o_ref[...] = xc * lax.rsqrt(var + 1e-5) * g_ref[...] + bb_ref[...]


DB = 512


def _dense(wsum, den, nrep, W, b, g, bb):
    grid = NP // DB
    return pl.pallas_call(
        _dense_body,
        grid=(grid,),
        in_specs=[
            pl.BlockSpec((NC, DB, N_HID), lambda i: (0, i, 0)),
            pl.BlockSpec((NC, DB, 1), lambda i: (0, i, 0)),
            pl.BlockSpec((DB, N_HID), lambda i: (i, 0)),
            pl.BlockSpec((N_HID, N_HID), lambda i: (0, 0)),
            pl.BlockSpec((1, N_HID), lambda i: (0, 0)),
            pl.BlockSpec((1, N_HID), lambda i: (0, 0)),
            pl.BlockSpec((1, N_HID), lambda i: (0, 0)),
        ],
        out_specs=pl.BlockSpec((DB, N_HID), lambda i: (i, 0)),
        out_shape=jax.ShapeDtypeStruct((NP, N_HID), jnp.float32),
    )(wsum, den, nrep, W, b, g, bb)


def _pool_body(x_ref, b_ref, W_ref, ob_ref, o_ref, acc, cnt):
    i = pl.program_id(0)

    @pl.when(i == 0)
    def _():
        acc[...] = jnp.zeros_like(acc)
        cnt[...] = jnp.zeros_like(cnt)

    gids = lax.broadcasted_iota(jnp.int32, (N_GRAPHS, DB), 0)
    onehot = (gids == b_ref[0]).astype(jnp.float32)
    acc[...] += jnp.dot(onehot, x_ref[...], preferred_element_type=jnp.float32)
    cnt[...] += jnp.sum(onehot, axis=1, keepdims=True)

    @pl.when(i == pl.num_programs(0) - 1)
    def _():
        pooled = acc[...] / jnp.maximum(cnt[...], 1.0)
        o_ref[...] = jnp.dot(pooled, W_ref[...],
                             preferred_element_type=jnp.float32) + ob_ref[...]


def _pool(nrep, batch2d, out_W, out_b):
    grid = NP // DB
    return pl.pallas_call(
        _pool_body,
        grid=(grid,),
        in_specs=[
            pl.BlockSpec((DB, N_HID), lambda i: (i, 0)),
            pl.BlockSpec((1, 1, DB), lambda i: (i, 0, 0)),
            pl.BlockSpec((N_HID, N_OUT), lambda i: (0, 0)),
            pl.BlockSpec((1, N_OUT), lambda i: (0, 0)),
        ],
        out_specs=pl.BlockSpec((N_GRAPHS, N_OUT), lambda i: (0, 0)),
        out_shape=jax.ShapeDtypeStruct((N_GRAPHS, N_OUT), jnp.float32),
        scratch_shapes=[
            pltpu.VMEM((N_GRAPHS, N_HID), jnp.float32),
            pltpu.VMEM((N_GRAPHS, 1), jnp.float32),
        ],
    )(nrep, batch2d, out_W, out_b)


def kernel(node_attr, edge_index, batch_idx, adv_atts, atom_emb, a_lin_W,
           a_lin_b, ln_g, ln_b, out_W, out_b):
    pad_attr = (jnp.arange(NP - N_NODES, dtype=jnp.int32)[None, :]
                + 7 * jnp.arange(ATOM_FEATS, dtype=jnp.int32)[:, None]) % 100
    attr_c = (jnp.concatenate(
        [node_attr.astype(jnp.int32).T, pad_attr], axis=1)
              .reshape(ATOM_FEATS, NP // NODE_CHUNK, NODE_CHUNK)
              .transpose(1, 0, 2))
    flat_emb = atom_emb.reshape(ATOM_FEATS * ATOM_VOCAB, N_HID)
    pad_idx = jnp.arange(EP - N_EDGES, dtype=jnp.int32) % N_NODES
    src = jnp.concatenate([edge_index[0].astype(jnp.int32), pad_idx])
    dst = jnp.concatenate([edge_index[1].astype(jnp.int32), pad_idx])
    a_p = jnp.pad(adv_atts, ((0, 0), (0, EP - N_EDGES)))
    edatas = [
        jnp.stack([src.reshape(NG, K), dst.reshape(NG, K),
                   lax.bitcast_convert_type(a_p[l], jnp.int32).reshape(NG, K)],
                  axis=1)
        for l in range(N_LAYERS)
    ]
    batch2d = jnp.pad(batch_idx.astype(jnp.int32), (0, NP - N_NODES),
                      constant_values=N_GRAPHS).reshape(NP // DB, 1, DB)

    def bf_shuffled(x):
        return lax.bitcast_convert_type(
            x.astype(jnp.bfloat16)
            .reshape(NP, N_HID // 32, 2, 16)
            .transpose(0, 1, 3, 2)
            .reshape(NP, N_HID // 2, 2), jnp.int32)

    nrep = _encoder(flat_emb, attr_c)
    for l in range(N_LAYERS):
        wsum, den = _spmm(bf_shuffled(nrep), edatas[l])
        nrep = _dense(wsum, den.reshape(NC, NP, 1), nrep, a_lin_W[l],
                      a_lin_b[l].reshape(1, N_HID), ln_g[l].reshape(1, N_HID),
                      ln_b[l].reshape(1, N_HID))
    return _pool(nrep, batch2d, out_W, out_b.reshape(1, N_OUT))


# speedup vs baseline: 1.3580x; 1.3580x over previous
import dataclasses
import functools

import jax
import jax.numpy as jnp
from jax import lax
from jax.experimental import pallas as pl
from jax.experimental.pallas import tpu as pltpu
from jax.experimental.pallas import tpu_sc as plsc

N_NODES = 10000
N_EDGES = 320000
N_HID = 128
N_OUT = 64
N_LAYERS = 2
N_GRAPHS = 64
ATOM_FEATS = 9
ATOM_VOCAB = 119

NC, NS = 2, 16
NW = NC * NS
NP = 10240
NPW = NP // NW
NPA = 10112
RPT = NPA // NS
K = 64
CPW = 160
EP = NW * CPW * K
NG = EP // K
CPW0 = CPW
CPW1 = 2 * CPW - CPW0
NODE_CHUNK = 64
ENC_CHUNKS = NPW // NODE_CHUNK


def _mesh():
    return plsc.VectorSubcoreMesh(core_axis_name="c", subcore_axis_name="s")


def _sc_params(**kw):
    cp = pltpu.CompilerParams()
    if "needs_layout_passes" in pltpu.CompilerParams.__dataclass_fields__:
        cp = dataclasses.replace(cp, needs_layout_passes=False)
    for k, v in kw.items():
        if k in pltpu.CompilerParams.__dataclass_fields__:
            cp = dataclasses.replace(cp, **{k: v})
    return cp


def _encoder(flat_emb, attr_c):
    @functools.partial(
        pl.kernel,
        mesh=_mesh(),
        out_type=jax.ShapeDtypeStruct((NP, N_HID), jnp.float32),
        scratch_types=[
            pltpu.VMEM((ATOM_FEATS, NODE_CHUNK), jnp.int32),
            pltpu.VMEM((ATOM_FEATS, NODE_CHUNK, N_HID), jnp.float32),
            pltpu.VMEM((NODE_CHUNK, N_HID), jnp.float32),
            pltpu.SemaphoreType.DMA,
        ],
        compiler_params=_sc_params(),
    )
    def enc(emb_hbm, attr_hbm, out_hbm, ibuf, rbuf, acc, sem):
        cid = lax.axis_index("c")
        sid = lax.axis_index("s")
        wid = sid * NC + cid
        gbase = wid * ENC_CHUNKS

        @pl.loop(0, ENC_CHUNKS)
        def _(c):
            pltpu.sync_copy(attr_hbm.at[gbase + c], ibuf)
            for f in range(1, ATOM_FEATS):
                for t in range(NODE_CHUNK // 16):
                    sl = pl.ds(t * 16, 16)
                    ibuf[f, sl] = ibuf[f, sl] + (ATOM_VOCAB * f)
            for f in range(ATOM_FEATS):
                pltpu.async_copy(emb_hbm.at[ibuf.at[f]], rbuf.at[f], sem)
            for f in range(ATOM_FEATS):
                pltpu.make_async_copy(emb_hbm.at[ibuf.at[f]], rbuf.at[f],
                                      sem).wait()

            @pl.loop(0, NODE_CHUNK)
            def _(r):
                for j in range(N_HID // 16):
                    sl = pl.ds(j * 16, 16)
                    s = rbuf[0, r, sl]
                    for f in range(1, ATOM_FEATS):
                        s = s + rbuf[f, r, sl]
                    acc[r, sl] = s

            pltpu.sync_copy(
                acc, out_hbm.at[pl.ds((gbase + c) * NODE_CHUNK, NODE_CHUNK)])

    return enc(flat_emb, attr_c)


def _spmm(nrep_bf, edata):
    @functools.partial(
        pl.kernel,
        mesh=_mesh(),
        out_type=(
            jax.ShapeDtypeStruct((NC, NP, N_HID), jnp.float32),
            jax.ShapeDtypeStruct((NC, NP), jnp.float32),
        ),
        scratch_types=[
            pltpu.VMEM((3, K), jnp.int32),
            pltpu.VMEM((3, K), jnp.int32),
            pltpu.VMEM((3, K), jnp.int32),
            pltpu.VMEM((3, K), jnp.int32),
            pltpu.VMEM((K,), jnp.float32),
            pltpu.VMEM((K,), jnp.float32),
            pltpu.VMEM((K,), jnp.float32),
            pltpu.VMEM((K,), jnp.float32),
            pltpu.VMEM((K, N_HID // 2), jnp.int32),
            pltpu.VMEM((K, N_HID // 2), jnp.int32),
            pltpu.VMEM((K, N_HID), jnp.float32),
            pltpu.VMEM((K, N_HID), jnp.float32),
            pltpu.VMEM_SHARED((NPA, N_HID), jnp.float32),
            pltpu.VMEM_SHARED((NPA,), jnp.float32),
            pltpu.SemaphoreType.DMA,
            pltpu.SemaphoreType.DMA,
            pltpu.SemaphoreType.DMA,
            pltpu.SemaphoreType.DMA,
            pltpu.SemaphoreType.DMA,
            pltpu.SemaphoreType.DMA,
            pltpu.SemaphoreType.DMA,
            pltpu.SemaphoreType.DMA,
        ],
        compiler_params=_sc_params(use_tc_tiling_on_sc=False),
    )
    def spmm(nrep_hbm, edata_hbm, wsum_hbm, den_hbm,
             ebuf0, ebuf1, ebuf2, ebuf3, av0, av1, av2, av3,
             rb0, rb1, rows0, rows1, wsum_sh, den_sh,
             semg0, semg1, sems0, sems1, semi0, semi1, semi2, semi3):
        cid = lax.axis_index("c")
        sid = lax.axis_index("s")
        is0 = cid == 0
        gbase = jnp.where(is0, sid * CPW0, NS * CPW0 + sid * CPW1)
        nch = jnp.where(is0, CPW0, CPW1)

        ebufs = (ebuf0, ebuf1, ebuf2, ebuf3)
        avs = (av0, av1, av2, av3)
        rbs = (rb0, rb1)
        rows = (rows0, rows1)
        semgs = (semg0, semg1)
        semss = (sems0, sems1)
        semis = (semi0, semi1, semi2, semi3)

        scope_zero = jax.named_scope("spmm_zero")
        scope_zero.__enter__()
        zero16 = jnp.zeros((16,), jnp.float32)

        @pl.loop(0, K)
        def _(r):
            for j in range(N_HID // 16):
                rows0[r, pl.ds(j * 16, 16)] = zero16

        for j in range(K // 16):
            av0[pl.ds(j * 16, 16)] = zero16

        stripe = sid * RPT

        for c in range(10):
            sz = K if c < 9 else RPT - 9 * K
            pltpu.sync_copy(rows0.at[pl.ds(0, sz)],
                            wsum_sh.at[pl.ds(stripe + c * K, sz)])
            pltpu.sync_copy(av0.at[pl.ds(0, sz)],
                            den_sh.at[pl.ds(stripe + c * K, sz)])

        @pl.when(sid == NS - 1)
        def _():
            for c in range((NP - NPA) // K):
                pltpu.sync_copy(rows0,
                                wsum_hbm.at[cid, pl.ds(NPA + c * K, K)])
                pltpu.sync_copy(av0, den_hbm.at[cid, pl.ds(NPA + c * K, K)])

        plsc.subcore_barrier()
        scope_zero.__exit__(None, None, None)

        def idx_start(e, g):
            pltpu.async_copy(edata_hbm.at[g], ebufs[e], semis[e])

        def idx_wait(e):
            pltpu.make_async_copy(edata_hbm.at[0], ebufs[e], semis[e]).wait()

        def build_av(e):
            for j in range(K // 16):
                sl = pl.ds(j * 16, 16)
                avs[e][sl] = plsc.bitcast(ebufs[e][2, sl], jnp.float32)

        def gather_start(p, e):
            pltpu.async_copy(nrep_hbm.at[ebufs[e].at[0]], rbs[p], semgs[p])

        def gather_wait(p, e):
            pltpu.make_async_copy(nrep_hbm.at[ebufs[e].at[0]], rbs[p],
                                  semgs[p]).wait()

        def scatter_start(p, e):
            pltpu.async_copy(rows[p], wsum_sh.at[ebufs[e].at[1]], semss[p],
                             add=True)
            pltpu.async_copy(avs[e], den_sh.at[ebufs[e].at[1]], semss[p],
                             add=True)

        def scatter_wait(p, e):
            pltpu.make_async_copy(rows[p], wsum_sh.at[ebufs[e].at[1]],
                                  semss[p]).wait()
            pltpu.make_async_copy(avs[e], den_sh.at[ebufs[e].at[1]],
                                  semss[p]).wait()

        himask = jnp.full((16,), -65536, jnp.int32)

        def scale(p, e):
            @pl.loop(0, K)
            def _(k):
                vs = plsc.load_gather(avs[e], [jnp.full((16,), k, jnp.int32)])
                for g in range(N_HID // 32):
                    w = rbs[p][k, pl.ds(g * 16, 16)]
                    lo = plsc.bitcast(w << 16, jnp.float32) * vs
                    hi = plsc.bitcast(w & himask, jnp.float32) * vs
                    rows[p][k, pl.ds(g * 32, 16)] = lo
                    rows[p][k, pl.ds(g * 32 + 16, 16)] = hi

        scope_loop = jax.named_scope("spmm_loop")
        scope_loop.__enter__()
        idx_start(0, gbase)
        idx_start(1, gbase + 1)
        idx_wait(0)
        build_av(0)
        gather_start(0, 0)

        def body(i, carry):
            c0 = i * 4
            for j in range(4):
                p, e, c = j & 1, j, c0 + j
                gather_wait(p, e)

                @pl.when(c >= 2)
                def _():
                    scatter_wait(p, (j + 2) % 4)

                @pl.when(c + 2 < nch)
                def _():
                    idx_start((j + 2) % 4, gbase + c + 2)

                @pl.when(c + 1 < nch)
                def _():
                    idx_wait((j + 1) % 4)
                    build_av((j + 1) % 4)
                    gather_start(1 - p, (j + 1) % 4)

                scale(p, e)
                scatter_start(p, e)
            return carry

        lax.fori_loop(0, nch // 4, body, 0)

        scatter_wait(0, 2)
        scatter_wait(1, 3)
        plsc.subcore_barrier()
        scope_loop.__exit__(None, None, None)

        scope_out = jax.named_scope("spmm_out")
        scope_out.__enter__()

        for c in range(10):
            sz = K if c < 9 else RPT - 9 * K
            off = stripe + c * K
            pltpu.sync_copy(wsum_sh.at[pl.ds(off, sz)],
                            wsum_hbm.at[cid, pl.ds(off, sz)])
            pltpu.sync_copy(den_sh.at[pl.ds(off, sz)],
                            den_hbm.at[cid, pl.ds(off, sz)])

        scope_out.__exit__(None, None, None)

    return spmm(nrep_bf, edata)


def _dense_body(w_ref, d_ref, x_ref, W_ref, b_ref, g_ref, bb_ref, o_ref):
    ws = w_ref[0] + w_ref[1]
    den = d_ref[0] + d_ref[1]
    aggr = ws * (1.0 / jnp.maximum(den, 1e-30))
    h = jnp.dot(jnp.maximum(aggr, 0.0), W_ref[...],
                preferred_element_type=jnp.float32) + b_ref[...]
    x = h + x_ref[...]
    mean = jnp.mean(x, axis=1, keepdims=True)
    xc = x - mean
    var = jnp.mean(xc * xc, axis=1, keepdims=True)
    o_ref[...] = xc * lax.rsqrt(var + 1e-5) * g_ref[...] + bb_ref[...]


DB = 512


def _dense(wsum, den, nrep, W, b, g, bb):
    grid = NP // DB
    return pl.pallas_call(
        _dense_body,
        grid=(grid,),
        in_specs=[
            pl.BlockSpec((NC, DB, N_HID), lambda i: (0, i, 0)),
            pl.BlockSpec((NC, DB, 1), lambda i: (0, i, 0)),
            pl.BlockSpec((DB, N_HID), lambda i: (i, 0)),
            pl.BlockSpec((N_HID, N_HID), lambda i: (0, 0)),
            pl.BlockSpec((1, N_HID), lambda i: (0, 0)),
            pl.BlockSpec((1, N_HID), lambda i: (0, 0)),
            pl.BlockSpec((1, N_HID), lambda i: (0, 0)),
        ],
        out_specs=pl.BlockSpec((DB, N_HID), lambda i: (i, 0)),
        out_shape=jax.ShapeDtypeStruct((NP, N_HID), jnp.float32),
    )(wsum, den, nrep, W, b, g, bb)


def _pool_body(x_ref, b_ref, W_ref, ob_ref, o_ref, acc, cnt):
    i = pl.program_id(0)

    @pl.when(i == 0)
    def _():
        acc[...] = jnp.zeros_like(acc)
        cnt[...] = jnp.zeros_like(cnt)

    gids = lax.broadcasted_iota(jnp.int32, (N_GRAPHS, DB), 0)
    onehot = (gids == b_ref[0]).astype(jnp.float32)
    acc[...] += jnp.dot(onehot, x_ref[...], preferred_element_type=jnp.float32)
    cnt[...] += jnp.sum(onehot, axis=1, keepdims=True)

    @pl.when(i == pl.num_programs(0) - 1)
    def _():
        pooled = acc[...] / jnp.maximum(cnt[...], 1.0)
        o_ref[...] = jnp.dot(pooled, W_ref[...],
                             preferred_element_type=jnp.float32) + ob_ref[...]


def _pool(nrep, batch2d, out_W, out_b):
    grid = NP // DB
    return pl.pallas_call(
        _pool_body,
        grid=(grid,),
        in_specs=[
            pl.BlockSpec((DB, N_HID), lambda i: (i, 0)),
            pl.BlockSpec((1, 1, DB), lambda i: (i, 0, 0)),
            pl.BlockSpec((N_HID, N_OUT), lambda i: (0, 0)),
            pl.BlockSpec((1, N_OUT), lambda i: (0, 0)),
        ],
        out_specs=pl.BlockSpec((N_GRAPHS, N_OUT), lambda i: (0, 0)),
        out_shape=jax.ShapeDtypeStruct((N_GRAPHS, N_OUT), jnp.float32),
        scratch_shapes=[
            pltpu.VMEM((N_GRAPHS, N_HID), jnp.float32),
            pltpu.VMEM((N_GRAPHS, 1), jnp.float32),
        ],
    )(nrep, batch2d, out_W, out_b)


def kernel(node_attr, edge_index, batch_idx, adv_atts, atom_emb, a_lin_W,
           a_lin_b, ln_g, ln_b, out_W, out_b):
    pad_attr = (jnp.arange(NP - N_NODES, dtype=jnp.int32)[None, :]
                + 7 * jnp.arange(ATOM_FEATS, dtype=jnp.int32)[:, None]) % 100
    attr_c = (jnp.concatenate(
        [node_attr.astype(jnp.int32).T, pad_attr], axis=1)
              .reshape(ATOM_FEATS, NP // NODE_CHUNK, NODE_CHUNK)
              .transpose(1, 0, 2))
    flat_emb = atom_emb.reshape(ATOM_FEATS * ATOM_VOCAB, N_HID)
    pad_idx = jnp.arange(EP - N_EDGES, dtype=jnp.int32) % N_NODES
    src = jnp.concatenate([edge_index[0].astype(jnp.int32), pad_idx])
    dst = jnp.concatenate([edge_index[1].astype(jnp.int32), pad_idx])
    a_p = jnp.pad(adv_atts, ((0, 0), (0, EP - N_EDGES)))
    edatas = [
        jnp.stack([src.reshape(NG, K), dst.reshape(NG, K),
                   lax.bitcast_convert_type(a_p[l], jnp.int32).reshape(NG, K)],
                  axis=1)
        for l in range(N_LAYERS)
    ]
    batch2d = jnp.pad(batch_idx.astype(jnp.int32), (0, NP - N_NODES),
                      constant_values=N_GRAPHS).reshape(NP // DB, 1, DB)

    def bf_shuffled(x):
        return lax.bitcast_convert_type(
            x.astype(jnp.bfloat16)
            .reshape(NP, N_HID // 32, 2, 16)
            .transpose(0, 1, 3, 2)
            .reshape(NP, N_HID // 2, 2), jnp.int32)

    nrep = _encoder(flat_emb, attr_c)
    for l in range(N_LAYERS):
        wsum, den = _spmm(bf_shuffled(nrep), edatas[l])
        nrep = _dense(wsum, den.reshape(NC, NP, 1), nrep, a_lin_W[l],
                      a_lin_b[l].reshape(1, N_HID), ln_g[l].reshape(1, N_HID),
                      ln_b[l].reshape(1, N_HID))
    return _pool(nrep, batch2d, out_W, out_b.reshape(1, N_OUT))


# speedup vs baseline: 1.3709x; 1.0095x over previous
import dataclasses
import functools

import jax
import jax.numpy as jnp
from jax import lax
from jax.experimental import pallas as pl
from jax.experimental.pallas import tpu as pltpu
from jax.experimental.pallas import tpu_sc as plsc

N_NODES = 10000
N_EDGES = 320000
N_HID = 128
N_OUT = 64
N_LAYERS = 2
N_GRAPHS = 64
ATOM_FEATS = 9
ATOM_VOCAB = 119

NC, NS = 2, 16
NW = NC * NS
NP = 10240
NPW = NP // NW
NPA = 10112
RPT = NPA // NS
K = 96
CPW = 108
EP = NW * CPW * K
NG = EP // K
CPW0 = CPW
CPW1 = 2 * CPW - CPW0
NODE_CHUNK = 64
ENC_CHUNKS = NPW // NODE_CHUNK


def _mesh():
    return plsc.VectorSubcoreMesh(core_axis_name="c", subcore_axis_name="s")


def _sc_params(**kw):
    cp = pltpu.CompilerParams()
    if "needs_layout_passes" in pltpu.CompilerParams.__dataclass_fields__:
        cp = dataclasses.replace(cp, needs_layout_passes=False)
    for k, v in kw.items():
        if k in pltpu.CompilerParams.__dataclass_fields__:
            cp = dataclasses.replace(cp, **{k: v})
    return cp


def _encoder(flat_emb, attr_c):
    @functools.partial(
        pl.kernel,
        mesh=_mesh(),
        out_type=jax.ShapeDtypeStruct((NP, N_HID), jnp.float32),
        scratch_types=[
            pltpu.VMEM((ATOM_FEATS, NODE_CHUNK), jnp.int32),
            pltpu.VMEM((ATOM_FEATS, NODE_CHUNK, N_HID), jnp.float32),
            pltpu.VMEM((NODE_CHUNK, N_HID), jnp.float32),
            pltpu.SemaphoreType.DMA,
        ],
        compiler_params=_sc_params(),
    )
    def enc(emb_hbm, attr_hbm, out_hbm, ibuf, rbuf, acc, sem):
        cid = lax.axis_index("c")
        sid = lax.axis_index("s")
        wid = sid * NC + cid
        gbase = wid * ENC_CHUNKS

        @pl.loop(0, ENC_CHUNKS)
        def _(c):
            pltpu.sync_copy(attr_hbm.at[gbase + c], ibuf)
            for f in range(1, ATOM_FEATS):
                for t in range(NODE_CHUNK // 16):
                    sl = pl.ds(t * 16, 16)
                    ibuf[f, sl] = ibuf[f, sl] + (ATOM_VOCAB * f)
            for f in range(ATOM_FEATS):
                pltpu.async_copy(emb_hbm.at[ibuf.at[f]], rbuf.at[f], sem)
            for f in range(ATOM_FEATS):
                pltpu.make_async_copy(emb_hbm.at[ibuf.at[f]], rbuf.at[f],
                                      sem).wait()

            @pl.loop(0, NODE_CHUNK)
            def _(r):
                for j in range(N_HID // 16):
                    sl = pl.ds(j * 16, 16)
                    s = rbuf[0, r, sl]
                    for f in range(1, ATOM_FEATS):
                        s = s + rbuf[f, r, sl]
                    acc[r, sl] = s

            pltpu.sync_copy(
                acc, out_hbm.at[pl.ds((gbase + c) * NODE_CHUNK, NODE_CHUNK)])

    return enc(flat_emb, attr_c)


def _spmm(nrep_bf, edata):
    @functools.partial(
        pl.kernel,
        mesh=_mesh(),
        out_type=(
            jax.ShapeDtypeStruct((NC, NP, N_HID), jnp.float32),
            jax.ShapeDtypeStruct((NC, NP), jnp.float32),
        ),
        scratch_types=[
            pltpu.VMEM((3, K), jnp.int32),
            pltpu.VMEM((3, K), jnp.int32),
            pltpu.VMEM((3, K), jnp.int32),
            pltpu.VMEM((3, K), jnp.int32),
            pltpu.VMEM((K,), jnp.float32),
            pltpu.VMEM((K,), jnp.float32),
            pltpu.VMEM((K,), jnp.float32),
            pltpu.VMEM((K,), jnp.float32),
            pltpu.VMEM((K, N_HID // 2), jnp.int32),
            pltpu.VMEM((K, N_HID // 2), jnp.int32),
            pltpu.VMEM((K, N_HID), jnp.float32),
            pltpu.VMEM((K, N_HID), jnp.float32),
            pltpu.VMEM_SHARED((NPA, N_HID), jnp.float32),
            pltpu.VMEM_SHARED((NPA,), jnp.float32),
            pltpu.SemaphoreType.DMA,
            pltpu.SemaphoreType.DMA,
            pltpu.SemaphoreType.DMA,
            pltpu.SemaphoreType.DMA,
            pltpu.SemaphoreType.DMA,
            pltpu.SemaphoreType.DMA,
            pltpu.SemaphoreType.DMA,
            pltpu.SemaphoreType.DMA,
        ],
        compiler_params=_sc_params(use_tc_tiling_on_sc=False),
    )
    def spmm(nrep_hbm, edata_hbm, wsum_hbm, den_hbm,
             ebuf0, ebuf1, ebuf2, ebuf3, av0, av1, av2, av3,
             rb0, rb1, rows0, rows1, wsum_sh, den_sh,
             semg0, semg1, sems0, sems1, semi0, semi1, semi2, semi3):
        cid = lax.axis_index("c")
        sid = lax.axis_index("s")
        is0 = cid == 0
        gbase = jnp.where(is0, sid * CPW0, NS * CPW0 + sid * CPW1)
        nch = jnp.where(is0, CPW0, CPW1)

        ebufs = (ebuf0, ebuf1, ebuf2, ebuf3)
        avs = (av0, av1, av2, av3)
        rbs = (rb0, rb1)
        rows = (rows0, rows1)
        semgs = (semg0, semg1)
        semss = (sems0, sems1)
        semis = (semi0, semi1, semi2, semi3)

        scope_zero = jax.named_scope("spmm_zero")
        scope_zero.__enter__()
        zero16 = jnp.zeros((16,), jnp.float32)

        @pl.loop(0, K)
        def _(r):
            for j in range(N_HID // 16):
                rows0[r, pl.ds(j * 16, 16)] = zero16

        for j in range(K // 16):
            av0[pl.ds(j * 16, 16)] = zero16

        stripe = sid * RPT

        zsizes = [K] * (RPT // K) + ([RPT % K] if RPT % K else [])
        for c, sz in enumerate(zsizes):
            pltpu.sync_copy(rows0.at[pl.ds(0, sz)],
                            wsum_sh.at[pl.ds(stripe + c * K, sz)])
            pltpu.sync_copy(av0.at[pl.ds(0, sz)],
                            den_sh.at[pl.ds(stripe + c * K, sz)])

        @pl.when(sid == NS - 1)
        def _():
            off = 0
            for sz in ([K] * ((NP - NPA) // K)
                       + ([(NP - NPA) % K] if (NP - NPA) % K else [])):
                pltpu.sync_copy(rows0.at[pl.ds(0, sz)],
                                wsum_hbm.at[cid, pl.ds(NPA + off, sz)])
                pltpu.sync_copy(av0.at[pl.ds(0, sz)],
                                den_hbm.at[cid, pl.ds(NPA + off, sz)])
                off += sz

        plsc.subcore_barrier()
        scope_zero.__exit__(None, None, None)

        def idx_start(e, g):
            pltpu.async_copy(edata_hbm.at[g], ebufs[e], semis[e])

        def idx_wait(e):
            pltpu.make_async_copy(edata_hbm.at[0], ebufs[e], semis[e]).wait()

        def build_av(e):
            for j in range(K // 16):
                sl = pl.ds(j * 16, 16)
                avs[e][sl] = plsc.bitcast(ebufs[e][2, sl], jnp.float32)

        def gather_start(p, e):
            pltpu.async_copy(nrep_hbm.at[ebufs[e].at[0]], rbs[p], semgs[p])

        def gather_wait(p, e):
            pltpu.make_async_copy(nrep_hbm.at[ebufs[e].at[0]], rbs[p],
                                  semgs[p]).wait()

        def scatter_start(p, e):
            pltpu.async_copy(rows[p], wsum_sh.at[ebufs[e].at[1]], semss[p],
                             add=True)
            pltpu.async_copy(avs[e], den_sh.at[ebufs[e].at[1]], semss[p],
                             add=True)

        def scatter_wait(p, e):
            pltpu.make_async_copy(rows[p], wsum_sh.at[ebufs[e].at[1]],
                                  semss[p]).wait()
            pltpu.make_async_copy(avs[e], den_sh.at[ebufs[e].at[1]],
                                  semss[p]).wait()

        himask = jnp.full((16,), -65536, jnp.int32)

        def scale(p, e):
            @pl.loop(0, K, step=2)
            def _(k):
                for u in range(2):
                    ku = k + u
                    vs = plsc.load_gather(avs[e],
                                          [jnp.full((16,), ku, jnp.int32)])
                    for g in range(N_HID // 32):
                        w = rbs[p][ku, pl.ds(g * 16, 16)]
                        lo = plsc.bitcast(w << 16, jnp.float32) * vs
                        hi = plsc.bitcast(w & himask, jnp.float32) * vs
                        rows[p][ku, pl.ds(g * 32, 16)] = lo
                        rows[p][ku, pl.ds(g * 32 + 16, 16)] = hi

        scope_loop = jax.named_scope("spmm_loop")
        scope_loop.__enter__()
        idx_start(0, gbase)
        idx_start(1, gbase + 1)
        idx_wait(0)
        build_av(0)
        gather_start(0, 0)

        def body(i, carry):
            c0 = i * 4
            for j in range(4):
                p, e, c = j & 1, j, c0 + j
                gather_wait(p, e)

                @pl.when(c >= 2)
                def _():
                    scatter_wait(p, (j + 2) % 4)

                @pl.when(c + 2 < nch)
                def _():
                    idx_start((j + 2) % 4, gbase + c + 2)

                @pl.when(c + 1 < nch)
                def _():
                    idx_wait((j + 1) % 4)
                    build_av((j + 1) % 4)
                    gather_start(1 - p, (j + 1) % 4)

                scale(p, e)
                scatter_start(p, e)
            return carry

        lax.fori_loop(0, nch // 4, body, 0)

        scatter_wait(0, 2)
        scatter_wait(1, 3)
        plsc.subcore_barrier()
        scope_loop.__exit__(None, None, None)

        scope_out = jax.named_scope("spmm_out")
        scope_out.__enter__()

        for c, sz in enumerate(zsizes):
            off = stripe + c * K
            pltpu.sync_copy(wsum_sh.at[pl.ds(off, sz)],
                            wsum_hbm.at[cid, pl.ds(off, sz)])
            pltpu.sync_copy(den_sh.at[pl.ds(off, sz)],
                            den_hbm.at[cid, pl.ds(off, sz)])

        scope_out.__exit__(None, None, None)

    return spmm(nrep_bf, edata)


def _dense_body(w_ref, d_ref, x_ref, W_ref, b_ref, g_ref, bb_ref, o_ref):
    ws = w_ref[0] + w_ref[1]
    den = d_ref[0] + d_ref[1]
    aggr = ws * (1.0 / jnp.maximum(den, 1e-30))
    h = jnp.dot(jnp.maximum(aggr, 0.0), W_ref[...],
                preferred_element_type=jnp.float32) + b_ref[...]
    x = h + x_ref[...]
    mean = jnp.mean(x, axis=1, keepdims=True)
    xc = x - mean
    var = jnp.mean(xc * xc, axis=1, keepdims=True)
    o_ref[...] = xc * lax.rsqrt(var + 1e-5) * g_ref[...] + bb_ref[...]


DB = 512


def _dense(wsum, den, nrep, W, b, g, bb):
    grid = NP // DB
    return pl.pallas_call(
        _dense_body,
        grid=(grid,),
        in_specs=[
            pl.BlockSpec((NC, DB, N_HID), lambda i: (0, i, 0)),
            pl.BlockSpec((NC, DB, 1), lambda i: (0, i, 0)),
            pl.BlockSpec((DB, N_HID), lambda i: (i, 0)),
            pl.BlockSpec((N_HID, N_HID), lambda i: (0, 0)),
            pl.BlockSpec((1, N_HID), lambda i: (0, 0)),
            pl.BlockSpec((1, N_HID), lambda i: (0, 0)),
            pl.BlockSpec((1, N_HID), lambda i: (0, 0)),
        ],
        out_specs=pl.BlockSpec((DB, N_HID), lambda i: (i, 0)),
        out_shape=jax.ShapeDtypeStruct((NP, N_HID), jnp.float32),
    )(wsum, den, nrep, W, b, g, bb)


def _pool_body(x_ref, b_ref, W_ref, ob_ref, o_ref, acc, cnt):
    i = pl.program_id(0)

    @pl.when(i == 0)
    def _():
        acc[...] = jnp.zeros_like(acc)
        cnt[...] = jnp.zeros_like(cnt)

    gids = lax.broadcasted_iota(jnp.int32, (N_GRAPHS, DB), 0)
    onehot = (gids == b_ref[0]).astype(jnp.float32)
    acc[...] += jnp.dot(onehot, x_ref[...], preferred_element_type=jnp.float32)
    cnt[...] += jnp.sum(onehot, axis=1, keepdims=True)

    @pl.when(i == pl.num_programs(0) - 1)
    def _():
        pooled = acc[...] / jnp.maximum(cnt[...], 1.0)
        o_ref[...] = jnp.dot(pooled, W_ref[...],
                             preferred_element_type=jnp.float32) + ob_ref[...]


def _pool(nrep, batch2d, out_W, out_b):
    grid = NP // DB
    return pl.pallas_call(
        _pool_body,
        grid=(grid,),
        in_specs=[
            pl.BlockSpec((DB, N_HID), lambda i: (i, 0)),
            pl.BlockSpec((1, 1, DB), lambda i: (i, 0, 0)),
            pl.BlockSpec((N_HID, N_OUT), lambda i: (0, 0)),
            pl.BlockSpec((1, N_OUT), lambda i: (0, 0)),
        ],
        out_specs=pl.BlockSpec((N_GRAPHS, N_OUT), lambda i: (0, 0)),
        out_shape=jax.ShapeDtypeStruct((N_GRAPHS, N_OUT), jnp.float32),
        scratch_shapes=[
            pltpu.VMEM((N_GRAPHS, N_HID), jnp.float32),
            pltpu.VMEM((N_GRAPHS, 1), jnp.float32),
        ],
    )(nrep, batch2d, out_W, out_b)


def kernel(node_attr, edge_index, batch_idx, adv_atts, atom_emb, a_lin_W,
           a_lin_b, ln_g, ln_b, out_W, out_b):
    pad_attr = (jnp.arange(NP - N_NODES, dtype=jnp.int32)[None, :]
                + 7 * jnp.arange(ATOM_FEATS, dtype=jnp.int32)[:, None]) % 100
    attr_c = (jnp.concatenate(
        [node_attr.astype(jnp.int32).T, pad_attr], axis=1)
              .reshape(ATOM_FEATS, NP // NODE_CHUNK, NODE_CHUNK)
              .transpose(1, 0, 2))
    flat_emb = atom_emb.reshape(ATOM_FEATS * ATOM_VOCAB, N_HID)
    pad_idx = jnp.arange(EP - N_EDGES, dtype=jnp.int32) % N_NODES
    src = jnp.concatenate([edge_index[0].astype(jnp.int32), pad_idx])
    dst = jnp.concatenate([edge_index[1].astype(jnp.int32), pad_idx])
    a_p = jnp.pad(adv_atts, ((0, 0), (0, EP - N_EDGES)))
    edatas = [
        jnp.stack([src.reshape(NG, K), dst.reshape(NG, K),
                   lax.bitcast_convert_type(a_p[l], jnp.int32).reshape(NG, K)],
                  axis=1)
        for l in range(N_LAYERS)
    ]
    batch2d = jnp.pad(batch_idx.astype(jnp.int32), (0, NP - N_NODES),
                      constant_values=N_GRAPHS).reshape(NP // DB, 1, DB)

    def bf_shuffled(x):
        return lax.bitcast_convert_type(
            x.astype(jnp.bfloat16)
            .reshape(NP, N_HID // 32, 2, 16)
            .transpose(0, 1, 3, 2)
            .reshape(NP, N_HID // 2, 2), jnp.int32)

    nrep = _encoder(flat_emb, attr_c)
    for l in range(N_LAYERS):
        wsum, den = _spmm(bf_shuffled(nrep), edatas[l])
        nrep = _dense(wsum, den.reshape(NC, NP, 1), nrep, a_lin_W[l],
                      a_lin_b[l].reshape(1, N_HID), ln_g[l].reshape(1, N_HID),
                      ln_b[l].reshape(1, N_HID))
    return _pool(nrep, batch2d, out_W, out_b.reshape(1, N_OUT))


# speedup vs baseline: 1.7373x; 1.2672x over previous
import dataclasses
import functools

import jax
import jax.numpy as jnp
from jax import lax
from jax.experimental import pallas as pl
from jax.experimental.pallas import tpu as pltpu
from jax.experimental.pallas import tpu_sc as plsc

N_NODES = 10000
N_EDGES = 320000
N_HID = 128
N_OUT = 64
N_LAYERS = 2
N_GRAPHS = 64
ATOM_FEATS = 9
ATOM_VOCAB = 119

NC, NS = 2, 16
NW = NC * NS
NP = 10240
NPW = NP // NW
RPT = NP // NS
K = 128
CPW = 80
EP = NW * CPW * K
NG = EP // K
NODE_CHUNK = 64
ENC_CHUNKS = NPW // NODE_CHUNK
DB = 512


def _mesh():
    return plsc.VectorSubcoreMesh(core_axis_name="c", subcore_axis_name="s")


def _sc_params():
    cp = pltpu.CompilerParams()
    if "needs_layout_passes" in pltpu.CompilerParams.__dataclass_fields__:
        cp = dataclasses.replace(cp, needs_layout_passes=False)
    return cp


def _encoder(flat_emb, attr_c):
    @functools.partial(
        pl.kernel,
        mesh=_mesh(),
        out_type=jax.ShapeDtypeStruct((NP, N_HID), jnp.float32),
        scratch_types=[
            pltpu.VMEM((ATOM_FEATS, NODE_CHUNK), jnp.int32),
            pltpu.VMEM((ATOM_FEATS, NODE_CHUNK, N_HID), jnp.float32),
            pltpu.VMEM((NODE_CHUNK, N_HID), jnp.float32),
            pltpu.SemaphoreType.DMA,
        ],
        compiler_params=_sc_params(),
    )
    def enc(emb_hbm, attr_hbm, out_hbm, ibuf, rbuf, acc, sem):
        cid = lax.axis_index("c")
        sid = lax.axis_index("s")
        wid = sid * NC + cid
        gbase = wid * ENC_CHUNKS

        @pl.loop(0, ENC_CHUNKS)
        def _(c):
            pltpu.sync_copy(attr_hbm.at[gbase + c], ibuf)
            for f in range(1, ATOM_FEATS):
                for t in range(NODE_CHUNK // 16):
                    sl = pl.ds(t * 16, 16)
                    ibuf[f, sl] = ibuf[f, sl] + (ATOM_VOCAB * f)
            for f in range(ATOM_FEATS):
                pltpu.async_copy(emb_hbm.at[ibuf.at[f]], rbuf.at[f], sem)
            for f in range(ATOM_FEATS):
                pltpu.make_async_copy(emb_hbm.at[ibuf.at[f]], rbuf.at[f],
                                      sem).wait()

            @pl.loop(0, NODE_CHUNK)
            def _(r):
                for j in range(N_HID // 16):
                    sl = pl.ds(j * 16, 16)
                    s = rbuf[0, r, sl]
                    for f in range(1, ATOM_FEATS):
                        s = s + rbuf[f, r, sl]
                    acc[r, sl] = s

            pltpu.sync_copy(
                acc, out_hbm.at[pl.ds((gbase + c) * NODE_CHUNK, NODE_CHUNK)])

    return enc(flat_emb, attr_c)


def _spmm(nrep, edata):
    @functools.partial(
        pl.kernel,
        mesh=_mesh(),
        out_type=(
            jax.ShapeDtypeStruct((NC, NP, N_HID), jnp.float32),
            jax.ShapeDtypeStruct((NC, NP), jnp.float32),
        ),
        scratch_types=[
            pltpu.VMEM((3, K), jnp.int32),
            pltpu.VMEM((3, K), jnp.int32),
            pltpu.VMEM((K,), jnp.float32),
            pltpu.VMEM((K,), jnp.float32),
            pltpu.VMEM((K, N_HID), jnp.float32),
            pltpu.VMEM((K, N_HID), jnp.float32),
            pltpu.VMEM_SHARED((NP, N_HID), jnp.float32),
            pltpu.VMEM_SHARED((NP,), jnp.float32),
            pltpu.SemaphoreType.DMA,
            pltpu.SemaphoreType.DMA,
            pltpu.SemaphoreType.DMA,
            pltpu.SemaphoreType.DMA,
        ],
        compiler_params=_sc_params(),
    )
    def spmm(nrep_hbm, edata_hbm, wsum_hbm, den_hbm,
             ebuf0, ebuf1, av0, av1, rows0, rows1,
             wsum_sh, den_sh, semg0, semg1, sems0, sems1):
        cid = lax.axis_index("c")
        sid = lax.axis_index("s")
        wid = sid * NC + cid
        gbase = wid * CPW

        ebufs = (ebuf0, ebuf1)
        avs = (av0, av1)
        rows = (rows0, rows1)
        semgs = (semg0, semg1)
        semss = (sems0, sems1)

        zero16 = jnp.zeros((16,), jnp.float32)

        @pl.loop(0, K)
        def _(r):
            for j in range(N_HID // 16):
                rows0[r, pl.ds(j * 16, 16)] = zero16

        for j in range(K // 16):
            av0[pl.ds(j * 16, 16)] = zero16

        stripe = sid * RPT

        @pl.loop(0, RPT // K)
        def _(c):
            pltpu.sync_copy(rows0, wsum_sh.at[pl.ds(stripe + c * K, K)])
            pltpu.sync_copy(av0, den_sh.at[pl.ds(stripe + c * K, K)])

        plsc.subcore_barrier()

        def load_idx_and_a(b, g):
            pltpu.sync_copy(edata_hbm.at[g], ebufs[b])
            for j in range(K // 16):
                sl = pl.ds(j * 16, 16)
                avs[b][sl] = plsc.bitcast(ebufs[b][2, sl], jnp.float32)

        def start_gather(b):
            pltpu.async_copy(nrep_hbm.at[ebufs[b].at[0]], rows[b], semgs[b])

        def wait_gather(b):
            pltpu.make_async_copy(nrep_hbm.at[ebufs[b].at[0]], rows[b],
                                  semgs[b]).wait()

        def start_scatter(b):
            pltpu.async_copy(rows[b], wsum_sh.at[ebufs[b].at[1]], semss[b],
                             add=True)
            pltpu.async_copy(avs[b], den_sh.at[ebufs[b].at[1]], semss[b],
                             add=True)

        def wait_scatter(b):
            pltpu.make_async_copy(rows[b], wsum_sh.at[ebufs[b].at[1]],
                                  semss[b]).wait()
            pltpu.make_async_copy(avs[b], den_sh.at[ebufs[b].at[1]],
                                  semss[b]).wait()

        def scale(b):
            @pl.loop(0, K)
            def _(k):
                vs = plsc.load_gather(avs[b], [jnp.full((16,), k, jnp.int32)])
                for j in range(N_HID // 16):
                    sl = pl.ds(j * 16, 16)
                    rows[b][k, sl] = rows[b][k, sl] * vs

        load_idx_and_a(0, gbase)
        start_gather(0)

        @pl.loop(0, CPW, step=2)
        def _(c):
            wait_gather(0)

            @pl.when(c >= 2)
            def _():
                wait_scatter(1)

            load_idx_and_a(1, gbase + c + 1)
            start_gather(1)
            scale(0)
            start_scatter(0)

            wait_gather(1)
            wait_scatter(0)

            @pl.when(c + 2 < CPW)
            def _():
                load_idx_and_a(0, gbase + c + 2)
                start_gather(0)

            scale(1)
            start_scatter(1)

        wait_scatter(1)
        plsc.subcore_barrier()

        @pl.loop(0, RPT // K)
        def _(c):
            off = stripe + c * K
            pltpu.sync_copy(wsum_sh.at[pl.ds(off, K)],
                            wsum_hbm.at[cid, pl.ds(off, K)])
            pltpu.sync_copy(den_sh.at[pl.ds(off, K)],
                            den_hbm.at[cid, pl.ds(off, K)])

    return spmm(nrep, edata)


def _dense_body(w_ref, d_ref, x_ref, W_ref, b_ref, g_ref, bb_ref, o_ref):
    ws = w_ref[0] + w_ref[1]
    den = d_ref[0] + d_ref[1]
    aggr = ws * (1.0 / jnp.maximum(den, 1e-30))
    h = jnp.dot(jnp.maximum(aggr, 0.0), W_ref[...],
                preferred_element_type=jnp.float32) + b_ref[...]
    x = h + x_ref[...]
    mean = jnp.mean(x, axis=1, keepdims=True)
    xc = x - mean
    var = jnp.mean(xc * xc, axis=1, keepdims=True)
    o_ref[...] = xc * lax.rsqrt(var + 1e-5) * g_ref[...] + bb_ref[...]


def _dense(wsum, den, nrep, W, b, g, bb):
    grid = NP // DB
    return pl.pallas_call(
        _dense_body,
        grid=(grid,),
        in_specs=[
            pl.BlockSpec((NC, DB, N_HID), lambda i: (0, i, 0)),
            pl.BlockSpec((NC, DB, 1), lambda i: (0, i, 0)),
            pl.BlockSpec((DB, N_HID), lambda i: (i, 0)),
            pl.BlockSpec((N_HID, N_HID), lambda i: (0, 0)),
            pl.BlockSpec((1, N_HID), lambda i: (0, 0)),
            pl.BlockSpec((1, N_HID), lambda i: (0, 0)),
            pl.BlockSpec((1, N_HID), lambda i: (0, 0)),
        ],
        out_specs=pl.BlockSpec((DB, N_HID), lambda i: (i, 0)),
        out_shape=jax.ShapeDtypeStruct((NP, N_HID), jnp.float32),
    )(wsum, den, nrep, W, b, g, bb)


def _pool_body(x_ref, b_ref, W_ref, ob_ref, o_ref, acc, cnt):
    i = pl.program_id(0)

    @pl.when(i == 0)
    def _():
        acc[...] = jnp.zeros_like(acc)
        cnt[...] = jnp.zeros_like(cnt)

    gids = lax.broadcasted_iota(jnp.int32, (N_GRAPHS, DB), 0)
    onehot = (gids == b_ref[0]).astype(jnp.float32)
    acc[...] += jnp.dot(onehot, x_ref[...], preferred_element_type=jnp.float32)
    cnt[...] += jnp.sum(onehot, axis=1, keepdims=True)

    @pl.when(i == pl.num_programs(0) - 1)
    def _():
        pooled = acc[...] / jnp.maximum(cnt[...], 1.0)
        o_ref[...] = jnp.dot(pooled, W_ref[...],
                             preferred_element_type=jnp.float32) + ob_ref[...]


def _pool(nrep, batch2d, out_W, out_b):
    grid = NP // DB
    return pl.pallas_call(
        _pool_body,
        grid=(grid,),
        in_specs=[
            pl.BlockSpec((DB, N_HID), lambda i: (i, 0)),
            pl.BlockSpec((1, 1, DB), lambda i: (i, 0, 0)),
            pl.BlockSpec((N_HID, N_OUT), lambda i: (0, 0)),
            pl.BlockSpec((1, N_OUT), lambda i: (0, 0)),
        ],
        out_specs=pl.BlockSpec((N_GRAPHS, N_OUT), lambda i: (0, 0)),
        out_shape=jax.ShapeDtypeStruct((N_GRAPHS, N_OUT), jnp.float32),
        scratch_shapes=[
            pltpu.VMEM((N_GRAPHS, N_HID), jnp.float32),
            pltpu.VMEM((N_GRAPHS, 1), jnp.float32),
        ],
    )(nrep, batch2d, out_W, out_b)


def kernel(node_attr, edge_index, batch_idx, adv_atts, atom_emb, a_lin_W,
           a_lin_b, ln_g, ln_b, out_W, out_b):
    pad_attr = (jnp.arange(NP - N_NODES, dtype=jnp.int32)[None, :]
                + 7 * jnp.arange(ATOM_FEATS, dtype=jnp.int32)[:, None]) % 100
    attr_c = (jnp.concatenate(
        [node_attr.astype(jnp.int32).T, pad_attr], axis=1)
              .reshape(ATOM_FEATS, NP // NODE_CHUNK, NODE_CHUNK)
              .transpose(1, 0, 2))
    flat_emb = atom_emb.reshape(ATOM_FEATS * ATOM_VOCAB, N_HID)
    pad_idx = jnp.arange(EP - N_EDGES, dtype=jnp.int32) % N_NODES
    src = jnp.concatenate([edge_index[0].astype(jnp.int32), pad_idx])
    dst = jnp.concatenate([edge_index[1].astype(jnp.int32), pad_idx])
    a_p = jnp.pad(adv_atts, ((0, 0), (0, EP - N_EDGES)))
    edatas = [
        jnp.stack([src.reshape(NG, K), dst.reshape(NG, K),
                   lax.bitcast_convert_type(a_p[l], jnp.int32).reshape(NG, K)],
                  axis=1)
        for l in range(N_LAYERS)
    ]
    batch2d = jnp.pad(batch_idx.astype(jnp.int32), (0, NP - N_NODES),
                      constant_values=N_GRAPHS).reshape(NP // DB, 1, DB)

    nrep = _encoder(flat_emb, attr_c)
    for l in range(N_LAYERS):
        wsum, den = _spmm(nrep, edatas[l])
        nrep = _dense(wsum, den.reshape(NC, NP, 1), nrep, a_lin_W[l],
                      a_lin_b[l].reshape(1, N_HID), ln_g[l].reshape(1, N_HID),
                      ln_b[l].reshape(1, N_HID))
    return _pool(nrep, batch2d, out_W, out_b.reshape(1, N_OUT))
